# baseline (speedup 1.0000x reference)
import functools

import jax
import jax.numpy as jnp
from jax import lax
from jax.experimental import pallas as pl
from jax.experimental.pallas import tpu as pltpu
from jax.experimental.pallas import tpu_sc as plsc

H = 256
W = 256
D = 768
LANES = 16
NC = 2
NS = 16
NW = NC * NS
HI = 160
RPW = (H - HI) // NW
JC = 32
NJ = W // JC
LG = D // LANES
PAD = 8

_mesh = plsc.VectorSubcoreMesh(core_axis_name="c", subcore_axis_name="s")


@functools.partial(
    pl.kernel,
    mesh=_mesh,
    out_type=jax.ShapeDtypeStruct(((H - HI) * W, D), jnp.float32),
    scratch_types=[
        pltpu.VMEM((PAD,), jnp.int32),
        pltpu.VMEM((JC,), jnp.int32),
        pltpu.VMEM((JC,), jnp.int32),
        pltpu.VMEM((PAD, D), jnp.float32),
        pltpu.VMEM((JC, D), jnp.float32),
        pltpu.VMEM((JC, D), jnp.float32),
        pltpu.VMEM((JC, D), jnp.float32),
        pltpu.VMEM((JC, D), jnp.float32),
        pltpu.SemaphoreType.DMA,
        pltpu.SemaphoreType.DMA,
        pltpu.SemaphoreType.DMA,
        pltpu.SemaphoreType.DMA,
    ],
)
def _sc_embed(rows_pad_hbm, cols_hbm, row_table, col_table, out_hbm,
              ridx_v, cidx0_v, cidx1_v, rowe_v, cole0_v, cole1_v,
              outb0_v, outb1_v, sem_row, sem_c, sem_o0, sem_o1):
    wid = lax.axis_index("s") * NC + lax.axis_index("c")
    rbase = wid * RPW

    pltpu.sync_copy(rows_pad_hbm.at[wid], ridx_v)
    row_cp = pltpu.make_async_copy(row_table.at[ridx_v], rowe_v, sem_row)
    row_cp.start()

    def col_gather(cidx_v, cole_v, cj):
        pltpu.sync_copy(cols_hbm.at[pl.ds(cj * JC, JC)], cidx_v)
        pltpu.make_async_copy(col_table.at[cidx_v], cole_v, sem_c).start()

    col_gather(cidx0_v, cole0_v, 0)
    row_cp.wait()

    halves = ((cidx0_v, cole0_v), (cidx1_v, cole1_v))
    bufs = ((outb0_v, sem_o0), (outb1_v, sem_o1))

    def make_chunk_pair_body(il):
        def chunk_pair_body(cjp, _):
            for half, (cidx_v, cole_v) in enumerate(halves):
                cj = cjp * 2 + half
                outb_v, sem_o = bufs[half]
                pltpu.make_async_copy(
                    col_table.at[cidx_v], cole_v, sem_c).wait()
                n_cidx, n_cole = halves[1 - half]

                if half == 0:
                    col_gather(n_cidx, n_cole, cj + 1)
                else:
                    @pl.when(cjp < NJ // 2 - 1)
                    def _():
                        col_gather(n_cidx, n_cole, cj + 1)

                    if il < RPW - 1:
                        @pl.when(cjp == NJ // 2 - 1)
                        def _():
                            col_gather(n_cidx, n_cole, 0)

                def wait_out():
                    pltpu.make_async_copy(
                        outb_v, out_hbm.at[pl.ds(0, JC)], sem_o).wait()

                if il == 0:
                    @pl.when(cjp > 0)
                    def _():
                        wait_out()
                else:
                    wait_out()

                rvs = [rowe_v[il, pl.ds(g * LANES, LANES)]
                       for g in range(LG)]

                def r_body(r, _):
                    for g in range(LG):
                        sl = pl.ds(g * LANES, LANES)
                        outb_v[r, sl] = cole_v[r, sl] + rvs[g]
                    return 0

                lax.fori_loop(0, JC, r_body, 0)
                out_start = (rbase + il) * W + cj * JC
                pltpu.make_async_copy(
                    outb_v, out_hbm.at[pl.ds(out_start, JC)],
                    sem_o).start()
            return 0

        return chunk_pair_body

    for il in range(RPW):
        lax.fori_loop(0, NJ // 2, make_chunk_pair_body(il), 0)

    pltpu.make_async_copy(outb0_v, out_hbm.at[pl.ds(0, JC)], sem_o0).wait()
    pltpu.make_async_copy(outb1_v, out_hbm.at[pl.ds(0, JC)], sem_o1).wait()


def _tc_body(rows_sm, gs_sm, row_ref, col_ref, colfix_ref, out_ref):
    gw = gs_sm[1]
    jio = lax.broadcasted_iota(jnp.int32, (W, 1), 0)
    col = jnp.where(jio >= gw, colfix_ref[0], col_ref[...])
    out_ref[...] = col + row_ref[0]


_tc_embed = pl.pallas_call(
    _tc_body,
    grid_spec=pltpu.PrefetchScalarGridSpec(
        num_scalar_prefetch=2,
        grid=(HI,),
        in_specs=[
            pl.BlockSpec((1, 1, D), lambda i, rows_sm, gs_sm: (rows_sm[i], 0, 0)),
            pl.BlockSpec((W, D), lambda i, rows_sm, gs_sm: (0, 0)),
            pl.BlockSpec((1, 1, D), lambda i, rows_sm, gs_sm: (gs_sm[1] - 1, 0, 0)),
        ],
        out_specs=pl.BlockSpec((W, D), lambda i, rows_sm, gs_sm: (i, 0)),
    ),
    out_shape=jax.ShapeDtypeStruct((HI * W, D), jnp.float32),
)


def kernel(grid_size, row_table, col_table):
    gh = jnp.asarray(grid_size[0], jnp.int32)
    gw = jnp.asarray(grid_size[1], jnp.int32)
    rows = jnp.minimum(jnp.arange(H, dtype=jnp.int32), gh - 1)
    cols = jnp.minimum(jnp.arange(W, dtype=jnp.int32), gw - 1)
    gs_arr = jnp.stack([gh, gw])
    rows_sc = rows[HI:].reshape(NW, RPW)
    rows_pad = jnp.concatenate(
        [rows_sc, jnp.broadcast_to(rows_sc[:, -1:], (NW, PAD - RPW))], axis=1)
    row_table3 = row_table.reshape(row_table.shape[0], 1, D)
    col_table3 = col_table.reshape(col_table.shape[0], 1, D)
    tc_out = _tc_embed(rows, gs_arr, row_table3, col_table, col_table3)
    sc_out = _sc_embed(rows_pad, cols, row_table, col_table)
    return jnp.concatenate([tc_out, sc_out], axis=0)


# speedup vs baseline: 1.0344x; 1.0344x over previous
import functools

import jax
import jax.numpy as jnp
from jax import lax
from jax.experimental import pallas as pl
from jax.experimental.pallas import tpu as pltpu
from jax.experimental.pallas import tpu_sc as plsc

H = 256
W = 256
D = 768
LANES = 16
NC = 2
NS = 16
NW = NC * NS
HI = 160
RPW = (H - HI) // NW
JC = 32
NJ = W // JC
LG = D // LANES
PAD = 8

_mesh = plsc.VectorSubcoreMesh(core_axis_name="c", subcore_axis_name="s")


@functools.partial(
    pl.kernel,
    mesh=_mesh,
    out_type=jax.ShapeDtypeStruct(((H - HI) * W, D), jnp.float32),
    scratch_types=[
        pltpu.VMEM((PAD,), jnp.int32),
        pltpu.VMEM((JC,), jnp.int32),
        pltpu.VMEM((JC,), jnp.int32),
        pltpu.VMEM((PAD, D), jnp.float32),
        pltpu.VMEM((JC, D), jnp.float32),
        pltpu.VMEM((JC, D), jnp.float32),
        pltpu.VMEM((JC, D), jnp.float32),
        pltpu.VMEM((JC, D), jnp.float32),
        pltpu.SemaphoreType.DMA,
        pltpu.SemaphoreType.DMA,
        pltpu.SemaphoreType.DMA,
        pltpu.SemaphoreType.DMA,
    ],
)
def _sc_embed(rows_pad_hbm, cols_hbm, row_table, col_table, out_hbm,
              ridx_v, cidx0_v, cidx1_v, rowe_v, cole0_v, cole1_v,
              outb0_v, outb1_v, sem_row, sem_c, sem_o0, sem_o1):
    wid = lax.axis_index("s") * NC + lax.axis_index("c")
    rbase = wid * RPW

    pltpu.sync_copy(rows_pad_hbm.at[wid], ridx_v)
    row_cp = pltpu.make_async_copy(row_table.at[ridx_v], rowe_v, sem_row)
    row_cp.start()

    def col_gather(cidx_v, cole_v, cj):
        pltpu.sync_copy(cols_hbm.at[pl.ds(cj * JC, JC)], cidx_v)
        pltpu.make_async_copy(col_table.at[cidx_v], cole_v, sem_c).start()

    col_gather(cidx0_v, cole0_v, 0)
    row_cp.wait()

    halves = ((cidx0_v, cole0_v), (cidx1_v, cole1_v))
    bufs = ((outb0_v, sem_o0), (outb1_v, sem_o1))

    def chunk_pair_body(cjp, _):
        for half, (cidx_v, cole_v) in enumerate(halves):
            cj = cjp * 2 + half
            pltpu.make_async_copy(
                col_table.at[cidx_v], cole_v, sem_c).wait()
            n_cidx, n_cole = halves[1 - half]

            if half == 0:
                col_gather(n_cidx, n_cole, cj + 1)
            else:
                @pl.when(cjp < NJ // 2 - 1)
                def _():
                    col_gather(n_cidx, n_cole, cj + 1)

            for il in range(RPW):
                p = (RPW * half + il) % 2
                outb_v, sem_o = bufs[p]

                def wait_out(outb_v=outb_v, sem_o=sem_o):
                    pltpu.make_async_copy(
                        outb_v, out_hbm.at[pl.ds(0, JC)], sem_o).wait()

                if half == 0 and il < 2:
                    @pl.when(cjp > 0)
                    def _():
                        wait_out()
                else:
                    wait_out()

                rvs = [rowe_v[il, pl.ds(g * LANES, LANES)]
                       for g in range(LG)]

                def r_body(r, _, outb_v=outb_v, cole_v=cole_v, rvs=rvs):
                    for g in range(LG):
                        sl = pl.ds(g * LANES, LANES)
                        outb_v[r, sl] = cole_v[r, sl] + rvs[g]
                    return 0

                lax.fori_loop(0, JC, r_body, 0)
                out_start = (rbase + il) * W + cj * JC
                pltpu.make_async_copy(
                    outb_v, out_hbm.at[pl.ds(out_start, JC)],
                    sem_o).start()
        return 0

    lax.fori_loop(0, NJ // 2, chunk_pair_body, 0)

    pltpu.make_async_copy(outb0_v, out_hbm.at[pl.ds(0, JC)], sem_o0).wait()
    pltpu.make_async_copy(outb1_v, out_hbm.at[pl.ds(0, JC)], sem_o1).wait()


def _tc_body(rows_sm, gs_sm, row_ref, col_ref, colfix_ref, out_ref):
    gw = gs_sm[1]
    jio = lax.broadcasted_iota(jnp.int32, (W, 1), 0)
    col = jnp.where(jio >= gw, colfix_ref[0], col_ref[...])
    out_ref[...] = col + row_ref[0]


_tc_embed = pl.pallas_call(
    _tc_body,
    grid_spec=pltpu.PrefetchScalarGridSpec(
        num_scalar_prefetch=2,
        grid=(HI,),
        in_specs=[
            pl.BlockSpec((1, 1, D), lambda i, rows_sm, gs_sm: (rows_sm[i], 0, 0)),
            pl.BlockSpec((W, D), lambda i, rows_sm, gs_sm: (0, 0)),
            pl.BlockSpec((1, 1, D), lambda i, rows_sm, gs_sm: (gs_sm[1] - 1, 0, 0)),
        ],
        out_specs=pl.BlockSpec((W, D), lambda i, rows_sm, gs_sm: (i, 0)),
    ),
    out_shape=jax.ShapeDtypeStruct((HI * W, D), jnp.float32),
)


def kernel(grid_size, row_table, col_table):
    gh = jnp.asarray(grid_size[0], jnp.int32)
    gw = jnp.asarray(grid_size[1], jnp.int32)
    rows = jnp.minimum(jnp.arange(H, dtype=jnp.int32), gh - 1)
    cols = jnp.minimum(jnp.arange(W, dtype=jnp.int32), gw - 1)
    gs_arr = jnp.stack([gh, gw])
    rows_sc = rows[HI:].reshape(NW, RPW)
    rows_pad = jnp.concatenate(
        [rows_sc, jnp.broadcast_to(rows_sc[:, -1:], (NW, PAD - RPW))], axis=1)
    row_table3 = row_table.reshape(row_table.shape[0], 1, D)
    col_table3 = col_table.reshape(col_table.shape[0], 1, D)
    tc_out = _tc_embed(rows, gs_arr, row_table3, col_table, col_table3)
    sc_out = _sc_embed(rows_pad, cols, row_table, col_table)
    return jnp.concatenate([tc_out, sc_out], axis=0)


# speedup vs baseline: 2.4712x; 2.3889x over previous
import functools

import jax
import jax.numpy as jnp
from jax import lax
from jax.experimental import pallas as pl
from jax.experimental.pallas import tpu as pltpu
from jax.experimental.pallas import tpu_sc as plsc

H = 256
W = 256
D = 768
LANES = 16
NC = 2
NS = 16
NW = NC * NS
RPW = H // NW
JC = 32
NJ = W // JC
LG = D // LANES

_mesh = plsc.VectorSubcoreMesh(core_axis_name="c", subcore_axis_name="s")


@functools.partial(
    pl.kernel,
    mesh=_mesh,
    out_type=jax.ShapeDtypeStruct((H * W, D), jnp.float32),
    scratch_types=[
        pltpu.VMEM((RPW,), jnp.int32),
        pltpu.VMEM((JC,), jnp.int32),
        pltpu.VMEM((JC,), jnp.int32),
        pltpu.VMEM((RPW, D), jnp.float32),
        pltpu.VMEM((JC, D), jnp.float32),
        pltpu.VMEM((JC, D), jnp.float32),
        pltpu.VMEM((JC, D), jnp.float32),
        pltpu.VMEM((JC, D), jnp.float32),
        pltpu.SemaphoreType.DMA,
        pltpu.SemaphoreType.DMA,
        pltpu.SemaphoreType.DMA,
        pltpu.SemaphoreType.DMA,
    ],
)
def _sc_embed(rows_hbm, cols_hbm, row_table, col_table, out_hbm,
              ridx_v, cidx0_v, cidx1_v, rowe_v, cole0_v, cole1_v,
              outb0_v, outb1_v, sem_row, sem_c, sem_o0, sem_o1):
    wid = lax.axis_index("s") * NC + lax.axis_index("c")
    rbase = wid * RPW

    pltpu.sync_copy(rows_hbm.at[pl.ds(rbase, RPW)], ridx_v)
    row_cp = pltpu.make_async_copy(row_table.at[ridx_v], rowe_v, sem_row)
    row_cp.start()

    def col_gather(cidx_v, cole_v, cj):
        pltpu.sync_copy(cols_hbm.at[pl.ds(cj * JC, JC)], cidx_v)
        pltpu.make_async_copy(col_table.at[cidx_v], cole_v, sem_c).start()

    col_gather(cidx0_v, cole0_v, 0)
    row_cp.wait()

    halves = ((cidx0_v, cole0_v), (cidx1_v, cole1_v))
    bufs = ((outb0_v, sem_o0), (outb1_v, sem_o1))

    def chunk_pair_body(cjp, _):
        for half, (cidx_v, cole_v) in enumerate(halves):
            cj = cjp * 2 + half
            pltpu.make_async_copy(
                col_table.at[cidx_v], cole_v, sem_c).wait()
            n_cidx, n_cole = halves[1 - half]

            @pl.when(cj < NJ - 1)
            def _():
                col_gather(n_cidx, n_cole, cj + 1)

            def pair_body(tp, _):
                for b, (outb_v, sem_o) in enumerate(bufs):
                    il = tp * 2 + b
                    first_use = (cj == 0) & (tp == 0) if half == 0 else None

                    def wait_out():
                        pltpu.make_async_copy(
                            outb_v, out_hbm.at[pl.ds(0, JC)], sem_o).wait()

                    if half == 0:
                        @pl.when((cjp > 0) | (tp > 0))
                        def _():
                            wait_out()
                    else:
                        wait_out()

                    rvs = [rowe_v[il, pl.ds(g * LANES, LANES)]
                           for g in range(LG)]

                    def r_body(r, _):
                        for g in range(LG):
                            sl = pl.ds(g * LANES, LANES)
                            outb_v[r, sl] = cole_v[r, sl] + rvs[g]
                        return 0

                    lax.fori_loop(0, JC, r_body, 0)
                    out_start = (rbase + il) * W + cj * JC
                    pltpu.make_async_copy(
                        outb_v, out_hbm.at[pl.ds(out_start, JC)],
                        sem_o).start()
                return 0

            lax.fori_loop(0, RPW // 2, pair_body, 0)
        return 0

    lax.fori_loop(0, NJ // 2, chunk_pair_body, 0)

    pltpu.make_async_copy(outb0_v, out_hbm.at[pl.ds(0, JC)], sem_o0).wait()
    pltpu.make_async_copy(outb1_v, out_hbm.at[pl.ds(0, JC)], sem_o1).wait()


def kernel(grid_size, row_table, col_table):
    gh = jnp.asarray(grid_size[0], jnp.int32)
    gw = jnp.asarray(grid_size[1], jnp.int32)
    rows = jnp.minimum(jnp.arange(H, dtype=jnp.int32), gh - 1)
    cols = jnp.minimum(jnp.arange(W, dtype=jnp.int32), gw - 1)
    return _sc_embed(rows, cols, row_table, col_table)
